# baseline (speedup 1.0000x reference)
import functools

import jax
import jax.numpy as jnp
from jax.experimental import pallas as pl

N = 10000
E = 320000
D_IN = 128
D_E = 16
D_OUT = 768
HEADS = 4
DH = D_OUT // HEADS

ROW_BLK = 1000
EDGE_BLK = 8000


def _proj_body(x_ref, wcat_ref, h_ref, s_ref):
    xb = x_ref[...]
    wcat = wcat_ref[...]
    h_ref[...] = jnp.dot(xb, wcat[:, :D_OUT], preferred_element_type=jnp.float32)
    s_ref[...] = jnp.dot(xb, wcat[:, D_OUT:], preferred_element_type=jnp.float32)


def _node_proj(x, wcat):
    return pl.pallas_call(
        _proj_body,
        grid=(N // ROW_BLK,),
        in_specs=[
            pl.BlockSpec((ROW_BLK, D_IN), lambda i: (i, 0)),
            pl.BlockSpec((D_IN, D_OUT + 8), lambda i: (0, 0)),
        ],
        out_specs=[
            pl.BlockSpec((ROW_BLK, D_OUT), lambda i: (i, 0)),
            pl.BlockSpec((ROW_BLK, 8), lambda i: (i, 0)),
        ],
        out_shape=[
            jax.ShapeDtypeStruct((N, D_OUT), jnp.float32),
            jax.ShapeDtypeStruct((N, 8), jnp.float32),
        ],
    )(x, wcat)


def _le_body(ea_ref, wae_ref, le_ref):
    le_ref[...] = jnp.dot(ea_ref[...], wae_ref[...],
                          preferred_element_type=jnp.float32)


def _edge_logits(edge_attr, wae):
    return pl.pallas_call(
        _le_body,
        grid=(E // EDGE_BLK,),
        in_specs=[
            pl.BlockSpec((EDGE_BLK, D_E), lambda i: (i, 0)),
            pl.BlockSpec((D_E, HEADS), lambda i: (0, 0)),
        ],
        out_specs=pl.BlockSpec((EDGE_BLK, HEADS), lambda i: (i, 0)),
        out_shape=jax.ShapeDtypeStruct((E, HEADS), jnp.float32),
    )(edge_attr, wae)


def _final_body(out1_ref, a_ref, wbig_ref, o_ref):
    pre = out1_ref[...] + jnp.dot(a_ref[...], wbig_ref[...],
                                  preferred_element_type=jnp.float32)
    o_ref[...] = jnp.where(pre > 0, pre, jnp.exp(jnp.minimum(pre, 0.0)) - 1.0)


def _final(out1, aacc, wbig):
    return pl.pallas_call(
        _final_body,
        grid=(N // ROW_BLK,),
        in_specs=[
            pl.BlockSpec((ROW_BLK, D_OUT), lambda i: (i, 0)),
            pl.BlockSpec((ROW_BLK, HEADS * D_E), lambda i: (i, 0)),
            pl.BlockSpec((HEADS * D_E, D_OUT), lambda i: (0, 0)),
        ],
        out_specs=pl.BlockSpec((ROW_BLK, D_OUT), lambda i: (i, 0)),
        out_shape=jax.ShapeDtypeStruct((N, D_OUT), jnp.float32),
    )(out1, aacc, wbig)


def kernel(x, edge_index, edge_attr, W, We, a_src, a_dst, a_e):
    src, dst = edge_index[0], edge_index[1]

    Wr = W.reshape(D_IN, HEADS, DH)
    wa_src = jnp.einsum('khd,hd->kh', Wr, a_src)
    wa_dst = jnp.einsum('khd,hd->kh', Wr, a_dst)
    wcat = jnp.concatenate([W, wa_src, wa_dst], axis=1)
    Wer = We.reshape(D_E, HEADS, DH)
    wae = jnp.einsum('khd,hd->kh', Wer, a_e)
    wbig = jnp.zeros((HEADS * D_E, D_OUT), dtype=W.dtype)
    for hh in range(HEADS):
        wbig = wbig.at[hh * D_E:(hh + 1) * D_E,
                       hh * DH:(hh + 1) * DH].set(Wer[:, hh, :])

    h, s = _node_proj(x, wcat)
    s_src, s_dst = s[:, :HEADS], s[:, HEADS:]
    le = _edge_logits(edge_attr, wae)

    logits = s_src[src] + s_dst[dst] + le
    logits = jnp.maximum(logits, 0.2 * logits)
    ex = jnp.exp(logits)
    denom = jax.ops.segment_sum(ex, dst, num_segments=N)
    alpha = ex / (denom[dst] + 1e-16)
    out1 = jax.ops.segment_sum(
        alpha[:, :, None] * h[src].reshape(E, HEADS, DH), dst, num_segments=N)
    aacc = jax.ops.segment_sum(
        alpha[:, :, None] * edge_attr[:, None, :], dst, num_segments=N)

    return _final(out1.reshape(N, D_OUT), aacc.reshape(N, HEADS * D_E), wbig)


# speedup vs baseline: 1.0131x; 1.0131x over previous
import functools

import jax
import jax.numpy as jnp
from jax import lax
from jax.experimental import pallas as pl
from jax.experimental.pallas import tpu as pltpu
from jax.experimental.pallas import tpu_sc as plsc

N = 10000
E = 320000
D_IN = 128
D_E = 16
D_OUT = 768
HEADS = 4
DH = D_OUT // HEADS

ROW_BLK = 1000
EDGE_BLK = 8000

NTILES = 32
RANGE = 79
NBUCKET = 128
NP = NBUCKET * RANGE
EB_CHUNK = 2000


def _edge_pass(src4, dst4, le_flat, s_src, s_dst):
    mesh = plsc.VectorSubcoreMesh(core_axis_name="c", subcore_axis_name="s")
    epw = E // NTILES
    nchunks = epw // EB_CHUNK
    nvreg = EB_CHUNK * HEADS // 16
    CL = EB_CHUNK * HEADS

    @functools.partial(
        pl.kernel,
        out_type=[jax.ShapeDtypeStruct((E * HEADS,), jnp.float32),
                  jax.ShapeDtypeStruct((NTILES * NP * HEADS,), jnp.float32)],
        mesh=mesh,
        compiler_params=pltpu.CompilerParams(needs_layout_passes=False),
        scratch_types=[
            pltpu.VMEM((CL,), jnp.int32),
            pltpu.VMEM((CL,), jnp.int32),
            pltpu.VMEM((CL,), jnp.float32),
            pltpu.VMEM((CL,), jnp.float32),
            pltpu.VMEM((CL,), jnp.float32),
            pltpu.VMEM((CL,), jnp.float32),
            pltpu.VMEM((NP * HEADS,), jnp.float32),
            pltpu.SemaphoreType.DMA,
            pltpu.SemaphoreType.DMA,
        ],
    )
    def body(src4_h, dst4_h, le_h, ssrc_h, sdst_h, ex_h, denomp_h,
             isb, idb, lev, gsv, gdv, exv, accv, sem1, sem2):
        wid = lax.axis_index("s") * 2 + lax.axis_index("c")
        zero16 = jnp.zeros((16,), jnp.float32)

        def zbody(i, _):
            accv[pl.ds(i * 16, 16)] = zero16
            return 0
        lax.fori_loop(0, NP * HEADS // 16, zbody, 0)

        base0 = wid * epw * HEADS

        def chunk(ci, _):
            cb = base0 + ci * CL
            pltpu.sync_copy(src4_h.at[pl.ds(cb, CL)], isb)
            pltpu.sync_copy(dst4_h.at[pl.ds(cb, CL)], idb)
            pltpu.sync_copy(le_h.at[pl.ds(cb, CL)], lev)
            cp1 = pltpu.async_copy(ssrc_h.at[isb], gsv, sem1)
            cp2 = pltpu.async_copy(sdst_h.at[idb], gdv, sem2)
            cp1.wait()
            cp2.wait()

            def vb(t, _):
                sl = pl.ds(t * 16, 16)
                logit = gsv[sl] + gdv[sl] + lev[sl]
                logit = jnp.maximum(logit, 0.2 * logit)
                ev = jnp.exp(logit)
                exv[sl] = ev
                plsc.addupdate_scatter(accv, [idb[sl]], ev)
                return 0
            lax.fori_loop(0, nvreg, vb, 0)
            pltpu.sync_copy(exv, ex_h.at[pl.ds(cb, CL)])
            return 0
        lax.fori_loop(0, nchunks, chunk, 0)
        pltpu.sync_copy(accv, denomp_h.at[pl.ds(wid * NP * HEADS, NP * HEADS)])

    return body(src4, dst4, le_flat, s_src, s_dst)


def _proj_body(x_ref, wcat_ref, h_ref, s_ref):
    xb = x_ref[...]
    wcat = wcat_ref[...]
    h_ref[...] = jnp.dot(xb, wcat[:, :D_OUT], preferred_element_type=jnp.float32)
    s_ref[...] = jnp.dot(xb, wcat[:, D_OUT:], preferred_element_type=jnp.float32)


def _node_proj(x, wcat):
    return pl.pallas_call(
        _proj_body,
        grid=(N // ROW_BLK,),
        in_specs=[
            pl.BlockSpec((ROW_BLK, D_IN), lambda i: (i, 0)),
            pl.BlockSpec((D_IN, D_OUT + 8), lambda i: (0, 0)),
        ],
        out_specs=[
            pl.BlockSpec((ROW_BLK, D_OUT), lambda i: (i, 0)),
            pl.BlockSpec((ROW_BLK, 8), lambda i: (i, 0)),
        ],
        out_shape=[
            jax.ShapeDtypeStruct((N, D_OUT), jnp.float32),
            jax.ShapeDtypeStruct((N, 8), jnp.float32),
        ],
    )(x, wcat)


def _le_body(ea_ref, wae_ref, le_ref):
    le_ref[...] = jnp.dot(ea_ref[...], wae_ref[...],
                          preferred_element_type=jnp.float32)


def _edge_logits(edge_attr, wae):
    return pl.pallas_call(
        _le_body,
        grid=(E // EDGE_BLK,),
        in_specs=[
            pl.BlockSpec((EDGE_BLK, D_E), lambda i: (i, 0)),
            pl.BlockSpec((D_E, HEADS), lambda i: (0, 0)),
        ],
        out_specs=pl.BlockSpec((EDGE_BLK, HEADS), lambda i: (i, 0)),
        out_shape=jax.ShapeDtypeStruct((E, HEADS), jnp.float32),
    )(edge_attr, wae)


def _final_body(out1_ref, a_ref, wbig_ref, o_ref):
    pre = out1_ref[...] + jnp.dot(a_ref[...], wbig_ref[...],
                                  preferred_element_type=jnp.float32)
    o_ref[...] = jnp.where(pre > 0, pre, jnp.exp(jnp.minimum(pre, 0.0)) - 1.0)


def _final(out1, aacc, wbig):
    return pl.pallas_call(
        _final_body,
        grid=(N // ROW_BLK,),
        in_specs=[
            pl.BlockSpec((ROW_BLK, D_OUT), lambda i: (i, 0)),
            pl.BlockSpec((ROW_BLK, HEADS * D_E), lambda i: (i, 0)),
            pl.BlockSpec((HEADS * D_E, D_OUT), lambda i: (0, 0)),
        ],
        out_specs=pl.BlockSpec((ROW_BLK, D_OUT), lambda i: (i, 0)),
        out_shape=jax.ShapeDtypeStruct((N, D_OUT), jnp.float32),
    )(out1, aacc, wbig)


def kernel(x, edge_index, edge_attr, W, We, a_src, a_dst, a_e):
    src, dst = edge_index[0], edge_index[1]

    Wr = W.reshape(D_IN, HEADS, DH)
    wa_src = jnp.einsum('khd,hd->kh', Wr, a_src)
    wa_dst = jnp.einsum('khd,hd->kh', Wr, a_dst)
    wcat = jnp.concatenate([W, wa_src, wa_dst], axis=1)
    Wer = We.reshape(D_E, HEADS, DH)
    wae = jnp.einsum('khd,hd->kh', Wer, a_e)
    wbig = jnp.zeros((HEADS * D_E, D_OUT), dtype=W.dtype)
    for hh in range(HEADS):
        wbig = wbig.at[hh * D_E:(hh + 1) * D_E,
                       hh * DH:(hh + 1) * DH].set(Wer[:, hh, :])

    h, s = _node_proj(x, wcat)
    s_src, s_dst = s[:, :HEADS], s[:, HEADS:]
    le = _edge_logits(edge_attr, wae)

    lane4 = jnp.arange(HEADS, dtype=jnp.int32)
    src4 = (src[:, None] * HEADS + lane4).reshape(E * HEADS)
    dst4 = (dst[:, None] * HEADS + lane4).reshape(E * HEADS)
    ex_flat, denomp = _edge_pass(src4, dst4, le.reshape(E * HEADS),
                                 s_src.reshape(N * HEADS),
                                 s_dst.reshape(N * HEADS))
    ex_flat, denomp = jax.lax.optimization_barrier((ex_flat, denomp))
    ex = ex_flat.reshape(E, HEADS)
    denom = denomp.reshape(NTILES, NP, HEADS)[:, :N, :].sum(axis=0)
    alpha = ex / (denom[dst] + 1e-16)
    out1 = jax.ops.segment_sum(
        alpha[:, :, None] * h[src].reshape(E, HEADS, DH), dst, num_segments=N)
    aacc = jax.ops.segment_sum(
        alpha[:, :, None] * edge_attr[:, None, :], dst, num_segments=N)

    return _final(out1.reshape(N, D_OUT), aacc.reshape(N, HEADS * D_E), wbig)


# speedup vs baseline: 6.1258x; 6.0465x over previous
import functools

import jax
import jax.numpy as jnp
from jax import lax
from jax.experimental import pallas as pl
from jax.experimental.pallas import tpu as pltpu
from jax.experimental.pallas import tpu_sc as plsc

N = 10000
E = 320000
D_IN = 128
D_E = 16
D_OUT = 768
HEADS = 4
DH = D_OUT // HEADS

ROW_BLK = 1000
EDGE_BLK = 8000

NTILES = 32
RANGE = 80
NBUCKET = 125
PASSES = 4
NP = N
EB_CHUNK = 2000
SCANC = 2000
QCAP = SCANC + 16


def _edge_pass(src4, dst4, le_flat, s_src, s_dst):
    mesh = plsc.VectorSubcoreMesh(core_axis_name="c", subcore_axis_name="s")
    epw = E // NTILES
    nchunks = epw // EB_CHUNK
    nvreg = EB_CHUNK * HEADS // 16
    CL = EB_CHUNK * HEADS

    @functools.partial(
        pl.kernel,
        out_type=[jax.ShapeDtypeStruct((E * HEADS,), jnp.float32),
                  jax.ShapeDtypeStruct((NTILES * NP * HEADS,), jnp.float32)],
        mesh=mesh,
        compiler_params=pltpu.CompilerParams(needs_layout_passes=False),
        scratch_types=[
            pltpu.VMEM((CL,), jnp.int32),
            pltpu.VMEM((CL,), jnp.int32),
            pltpu.VMEM((CL,), jnp.float32),
            pltpu.VMEM((CL,), jnp.float32),
            pltpu.VMEM((CL,), jnp.float32),
            pltpu.VMEM((CL,), jnp.float32),
            pltpu.VMEM((NP * HEADS,), jnp.float32),
            pltpu.SemaphoreType.DMA,
            pltpu.SemaphoreType.DMA,
        ],
    )
    def body(src4_h, dst4_h, le_h, ssrc_h, sdst_h, ex_h, denomp_h,
             isb, idb, lev, gsv, gdv, exv, accv, sem1, sem2):
        wid = lax.axis_index("s") * 2 + lax.axis_index("c")
        zero16 = jnp.zeros((16,), jnp.float32)

        def zbody(i, _):
            accv[pl.ds(i * 16, 16)] = zero16
            return 0
        lax.fori_loop(0, NP * HEADS // 16, zbody, 0)

        base0 = wid * epw * HEADS

        def chunk(ci, _):
            cb = base0 + ci * CL
            pltpu.sync_copy(src4_h.at[pl.ds(cb, CL)], isb)
            pltpu.sync_copy(dst4_h.at[pl.ds(cb, CL)], idb)
            pltpu.sync_copy(le_h.at[pl.ds(cb, CL)], lev)
            cp1 = pltpu.async_copy(ssrc_h.at[isb], gsv, sem1)
            cp2 = pltpu.async_copy(sdst_h.at[idb], gdv, sem2)
            cp1.wait()
            cp2.wait()

            def vb(t, _):
                sl = pl.ds(t * 16, 16)
                logit = gsv[sl] + gdv[sl] + lev[sl]
                logit = jnp.maximum(logit, 0.2 * logit)
                ev = jnp.exp(logit)
                exv[sl] = ev
                plsc.addupdate_scatter(accv, [idb[sl]], ev)
                return 0
            lax.fori_loop(0, nvreg, vb, 0)
            pltpu.sync_copy(exv, ex_h.at[pl.ds(cb, CL)])
            return 0
        lax.fori_loop(0, nchunks, chunk, 0)
        pltpu.sync_copy(accv, denomp_h.at[pl.ds(wid * NP * HEADS, NP * HEADS)])

    return body(src4, dst4, le_flat, s_src, s_dst)


def _accum_pass(dst, src, ex_flat, denomp, h, edge_attr):
    mesh = plsc.VectorSubcoreMesh(core_axis_name="c", subcore_axis_name="s")
    nscan = E // SCANC
    nsvreg = SCANC // 16

    @functools.partial(
        pl.kernel,
        out_type=[jax.ShapeDtypeStruct((N * D_OUT,), jnp.float32),
                  jax.ShapeDtypeStruct((N * HEADS * D_E,), jnp.float32)],
        mesh=mesh,
        compiler_params=pltpu.CompilerParams(needs_layout_passes=False),
        scratch_types=[
            pltpu.VMEM((SCANC,), jnp.int32),
            pltpu.VMEM((SCANC,), jnp.int32),
            pltpu.VMEM((QCAP,), jnp.int32),
            pltpu.VMEM((QCAP,), jnp.int32),
            pltpu.VMEM((QCAP,), jnp.int32),
            pltpu.VMEM((336,), jnp.float32),
            pltpu.VMEM((NTILES * 320,), jnp.float32),
            pltpu.VMEM((RANGE * D_OUT,), jnp.float32),
            pltpu.VMEM((RANGE * HEADS * D_E,), jnp.float32),
            pltpu.VMEM((64,), jnp.int32),
            pltpu.VMEM((64,), jnp.float32),
            pltpu.VMEM((64,), jnp.float32),
            pltpu.VMEM((16, D_OUT), jnp.float32),
            pltpu.VMEM((256,), jnp.int32),
            pltpu.VMEM((256,), jnp.float32),
            pltpu.SemaphoreType.DMA,
            pltpu.SemaphoreType.DMA,
            pltpu.SemaphoreType.DMA,
        ],
    )
    def body(dst_h, src_h, ex_h, denomp_h, h_h, ea_h, out1_h, aacc_h,
             dstc, srcc, eidq, dlq, srcq, dloc, dtmp, accv, acca,
             exidx, exvals, albuf, hbuf, eaidx, eabuf, semh, semea, semex):
        wid = lax.axis_index("s") * 2 + lax.axis_index("c")
        iota = lax.iota(jnp.int32, 16)
        lane_r = iota >> 2
        lane_c = iota & 3
        zero16 = jnp.zeros((16,), jnp.float32)
        zero16i = jnp.zeros((16,), jnp.int32)

        def zq(i, _):
            sl = pl.ds(i * 16, 16)
            eidq[sl] = zero16i
            dlq[sl] = zero16i
            srcq[sl] = zero16i
            return 0
        lax.fori_loop(0, QCAP // 16, zq, 0)

        def one_pass(p, _):
            b = p * NTILES + wid

            @pl.when(b < NBUCKET)
            def _():
                lo = b * RANGE

                def zd(i, _):
                    dloc[pl.ds(i * 16, 16)] = zero16
                    return 0
                lax.fori_loop(0, 336 // 16, zd, 0)
                cps = []
                for t2 in range(NTILES):
                    cps.append(pltpu.async_copy(
                        denomp_h.at[pl.ds(t2 * (N * HEADS) + lo * HEADS,
                                          RANGE * HEADS)],
                        dtmp.at[pl.ds(t2 * 320, 320)], semh))
                for cp in cps:
                    cp.wait()

                def sd(i, _):
                    def sd2(v, _):
                        sl = pl.ds(v * 16, 16)
                        dloc[sl] = dloc[sl] + dtmp[pl.ds(i * 320 + v * 16, 16)]
                        return 0
                    lax.fori_loop(0, 20, sd2, 0)
                    return 0
                lax.fori_loop(0, NTILES, sd, 0)

                def za(i, _):
                    accv[pl.ds(i * 16, 16)] = zero16
                    return 0
                lax.fori_loop(0, RANGE * D_OUT // 16, za, 0)

                def zb(i, _):
                    acca[pl.ds(i * 16, 16)] = zero16
                    return 0
                lax.fori_loop(0, RANGE * HEADS * D_E // 16, zb, 0)

                def chunk(ci, _):
                    cb = ci * SCANC
                    pltpu.sync_copy(dst_h.at[pl.ds(cb, SCANC)], dstc)
                    pltpu.sync_copy(src_h.at[pl.ds(cb, SCANC)], srcc)

                    def scan(v, qn):
                        sl = pl.ds(v * 16, 16)
                        d = dstc[sl]
                        dl = d - lo
                        m = (dl >= 0) & (dl < RANGE)
                        eidv = cb + v * 16 + iota
                        qsl = pl.ds(qn, 16)
                        plsc.store_compressed(eidq.at[qsl], eidv, mask=m)
                        plsc.store_compressed(dlq.at[qsl], dl, mask=m)
                        plsc.store_compressed(srcq.at[qsl], srcc[sl], mask=m)
                        cnt = plsc.all_reduce_population_count(m)[0]
                        return qn + cnt
                    qn = lax.fori_loop(0, nsvreg, scan, 0)

                    tsl = pl.ds(qn, 16)
                    srcq[tsl] = wid * 300 + iota
                    eidq[tsl] = zero16i

                    nsub = (qn + 15) // 16
                    qnf = jnp.full((16,), qn, jnp.int32)

                    def sub(s, _):
                        s16v = s * 16
                        for v in range(4):
                            posv = jnp.full((16,), s16v + v * 4,
                                            jnp.int32) + lane_r
                            eidrep = plsc.load_gather(eidq, [posv])
                            exidx[pl.ds(v * 16, 16)] = eidrep * 4 + lane_c
                        for j2 in range(16):
                            eidrep = plsc.load_gather(
                                eidq, [jnp.full((16,), s16v + j2, jnp.int32)])
                            eaidx[pl.ds(j2 * 16, 16)] = eidrep * D_E + iota
                        cph = pltpu.async_copy(
                            h_h.at[srcq.at[pl.ds(s16v, 16)]], hbuf, semh)
                        cpe = pltpu.async_copy(ea_h.at[eaidx], eabuf, semea)
                        cpx = pltpu.async_copy(ex_h.at[exidx], exvals, semex)
                        cph.wait()
                        cpe.wait()
                        cpx.wait()
                        for v in range(4):
                            posv = jnp.full((16,), s16v + v * 4,
                                            jnp.int32) + lane_r
                            dlrep = plsc.load_gather(dlq, [posv])
                            dv = plsc.load_gather(dloc, [dlrep * 4 + lane_c])
                            al = exvals[pl.ds(v * 16, 16)] / (dv + 1e-16)
                            al = jnp.where(posv < qnf, al, 0.0)
                            albuf[pl.ds(v * 16, 16)] = al

                        def je(j, _):
                            dlrep = plsc.load_gather(
                                dlq, [jnp.full((16,), s16v + j, jnp.int32)])
                            dbase = dlrep * D_OUT + iota
                            dbasea = dlrep * (HEADS * D_E) + iota
                            eav = eabuf[pl.ds(j * 16, 16)]
                            for hh in range(HEADS):
                                asp = plsc.load_gather(
                                    albuf,
                                    [jnp.full((16,), j * 4 + hh, jnp.int32)])
                                for v in range(DH // 16):
                                    off = hh * DH + v * 16
                                    hv = hbuf[j, pl.ds(off, 16)]
                                    plsc.addupdate_scatter(
                                        accv, [dbase + off], hv * asp)
                                plsc.addupdate_scatter(
                                    acca, [dbasea + hh * D_E], eav * asp)
                            return 0
                        lax.fori_loop(0, 16, je, 0)
                        return 0
                    lax.fori_loop(0, nsub, sub, 0)
                    return 0
                lax.fori_loop(0, nscan, chunk, 0)

                pltpu.sync_copy(accv, out1_h.at[pl.ds(lo * D_OUT,
                                                      RANGE * D_OUT)])
                pltpu.sync_copy(acca, aacc_h.at[pl.ds(lo * HEADS * D_E,
                                                      RANGE * HEADS * D_E)])
            return 0
        lax.fori_loop(0, PASSES, one_pass, 0)

    return body(dst, src, ex_flat, denomp, h, edge_attr)


def _proj_body(x_ref, wcat_ref, h_ref, s_ref):
    xb = x_ref[...]
    wcat = wcat_ref[...]
    h_ref[...] = jnp.dot(xb, wcat[:, :D_OUT], preferred_element_type=jnp.float32)
    s_ref[...] = jnp.dot(xb, wcat[:, D_OUT:], preferred_element_type=jnp.float32)


def _node_proj(x, wcat):
    return pl.pallas_call(
        _proj_body,
        grid=(N // ROW_BLK,),
        in_specs=[
            pl.BlockSpec((ROW_BLK, D_IN), lambda i: (i, 0)),
            pl.BlockSpec((D_IN, D_OUT + 8), lambda i: (0, 0)),
        ],
        out_specs=[
            pl.BlockSpec((ROW_BLK, D_OUT), lambda i: (i, 0)),
            pl.BlockSpec((ROW_BLK, 8), lambda i: (i, 0)),
        ],
        out_shape=[
            jax.ShapeDtypeStruct((N, D_OUT), jnp.float32),
            jax.ShapeDtypeStruct((N, 8), jnp.float32),
        ],
    )(x, wcat)


def _le_body(ea_ref, wae_ref, le_ref):
    le_ref[...] = jnp.dot(ea_ref[...], wae_ref[...],
                          preferred_element_type=jnp.float32)


def _edge_logits(edge_attr, wae):
    return pl.pallas_call(
        _le_body,
        grid=(E // EDGE_BLK,),
        in_specs=[
            pl.BlockSpec((EDGE_BLK, D_E), lambda i: (i, 0)),
            pl.BlockSpec((D_E, HEADS), lambda i: (0, 0)),
        ],
        out_specs=pl.BlockSpec((EDGE_BLK, HEADS), lambda i: (i, 0)),
        out_shape=jax.ShapeDtypeStruct((E, HEADS), jnp.float32),
    )(edge_attr, wae)


def _final_body(out1_ref, a_ref, wbig_ref, o_ref):
    pre = out1_ref[...] + jnp.dot(a_ref[...], wbig_ref[...],
                                  preferred_element_type=jnp.float32)
    o_ref[...] = jnp.where(pre > 0, pre, jnp.exp(jnp.minimum(pre, 0.0)) - 1.0)


def _final(out1, aacc, wbig):
    return pl.pallas_call(
        _final_body,
        grid=(N // ROW_BLK,),
        in_specs=[
            pl.BlockSpec((ROW_BLK, D_OUT), lambda i: (i, 0)),
            pl.BlockSpec((ROW_BLK, HEADS * D_E), lambda i: (i, 0)),
            pl.BlockSpec((HEADS * D_E, D_OUT), lambda i: (0, 0)),
        ],
        out_specs=pl.BlockSpec((ROW_BLK, D_OUT), lambda i: (i, 0)),
        out_shape=jax.ShapeDtypeStruct((N, D_OUT), jnp.float32),
    )(out1, aacc, wbig)


def kernel(x, edge_index, edge_attr, W, We, a_src, a_dst, a_e):
    src, dst = edge_index[0], edge_index[1]

    Wr = W.reshape(D_IN, HEADS, DH)
    wa_src = jnp.einsum('khd,hd->kh', Wr, a_src)
    wa_dst = jnp.einsum('khd,hd->kh', Wr, a_dst)
    wcat = jnp.concatenate([W, wa_src, wa_dst], axis=1)
    Wer = We.reshape(D_E, HEADS, DH)
    wae = jnp.einsum('khd,hd->kh', Wer, a_e)
    wbig = jnp.zeros((HEADS * D_E, D_OUT), dtype=W.dtype)
    for hh in range(HEADS):
        wbig = wbig.at[hh * D_E:(hh + 1) * D_E,
                       hh * DH:(hh + 1) * DH].set(Wer[:, hh, :])

    h, s = _node_proj(x, wcat)
    s_src, s_dst = s[:, :HEADS], s[:, HEADS:]
    le = _edge_logits(edge_attr, wae)

    lane4 = jnp.arange(HEADS, dtype=jnp.int32)
    src4 = (src[:, None] * HEADS + lane4).reshape(E * HEADS)
    dst4 = (dst[:, None] * HEADS + lane4).reshape(E * HEADS)
    ex_flat, denomp = _edge_pass(src4, dst4, le.reshape(E * HEADS),
                                 s_src.reshape(N * HEADS),
                                 s_dst.reshape(N * HEADS))
    out1_flat, aacc_flat = _accum_pass(dst, src, ex_flat, denomp,
                                       h, edge_attr.reshape(E * D_E))

    return _final(out1_flat.reshape(N, D_OUT),
                  aacc_flat.reshape(N, HEADS * D_E), wbig)
